# Initial kernel scaffold; baseline (speedup 1.0000x reference)
import jax
import jax.numpy as jnp
from jax.experimental import pallas as pl


def kernel(phrase_feat, rel_feat, rel_conn_mat, target_id, W_rel, b_rel, W_sub, b_sub, W_obj, b_obj, W_phr, b_phr):
    raise NotImplementedError("write your pallas kernel here")


# speedup vs baseline: 20.9123x; 20.9123x over previous
import functools

import jax
import jax.numpy as jnp
from jax.experimental import pallas as pl

N = 256
D = 256
E = 4096
EB = 1024
EPS = 1e-06
F32 = jnp.float32


def _tables_body(phr_ref, wa_ref, wsa_ref, wb_ref, woa_ref, tsub_ref, tobj_ref):
    phr = phr_ref[...]
    tsub_ref[:, :D] = jnp.dot(phr, wa_ref[...], preferred_element_type=F32)
    tsub_ref[:, D:] = jnp.dot(phr, wsa_ref[...], preferred_element_type=F32)
    tobj_ref[:, :D] = jnp.dot(phr, wb_ref[...], preferred_element_type=F32)
    tobj_ref[:, D:] = jnp.dot(phr, woa_ref[...], preferred_element_type=F32)


def _edges_body(sub_ref, obj_ref, rel_ref, tsub_ref, tobj_ref, wc_ref, wsb_ref,
                wob_ref, brel_ref, bsub_ref, bobj_ref, upd_ref, atte_ref):
    sub = sub_ref[0, :]
    obj = obj_ref[0, :]
    iota_n = jax.lax.broadcasted_iota(jnp.int32, (EB, N), 1)
    oh_sub = (sub[:, None] == iota_n).astype(F32)
    oh_obj = (obj[:, None] == iota_n).astype(F32)
    gsub = jnp.dot(oh_sub, tsub_ref[...], preferred_element_type=F32)
    gobj = jnp.dot(oh_obj, tobj_ref[...], preferred_element_type=F32)
    upd = (gsub[:, :D] + gobj[:, :D] + brel_ref[...]
           + jnp.dot(rel_ref[...], wc_ref[...], preferred_element_type=F32))
    upd_ref[...] = upd
    ts = gsub[:, D:] + bsub_ref[...] + jnp.dot(upd, wsb_ref[...],
                                               preferred_element_type=F32)
    to = gobj[:, D:] + bobj_ref[...] + jnp.dot(upd, wob_ref[...],
                                               preferred_element_type=F32)
    atte_ref[0, :] = jnp.sum(ts * to, axis=1) * (1.0 / (D ** 0.5))


def _context_body(sub_ref, obj_ref, atte_ref, upd_ref, phr_ref, wpa_ref,
                  wpb_ref, bphr_ref, t_ref, out_ref):
    t = t_ref[0, 0]
    sub = sub_ref[0, :]
    obj = obj_ref[0, :]
    atte = atte_ref[0, :]
    iota_e = jax.lax.broadcasted_iota(jnp.int32, (E, N), 0)
    iota_o = jax.lax.broadcasted_iota(jnp.int32, (E, N), 1)
    subc = sub[:, None]
    objc = obj[:, None]
    e_row = jnp.max(jnp.where((subc == t) & (objc == iota_o), iota_e, -1), axis=0)
    e_col = jnp.max(jnp.where((objc == t) & (subc == iota_o), iota_e, -1), axis=0)
    sel_row = iota_e == e_row[None, :]
    sel_col = iota_e == e_col[None, :]
    a_row = jnp.sum(jnp.where(sel_row, atte[:, None], 0.0), axis=0)
    a_col = jnp.sum(jnp.where(sel_col, atte[:, None], 0.0), axis=0)
    mask_row = (e_row >= 0).astype(F32)
    mask_col = (e_col >= 0).astype(F32)

    def msm(vec, mask):
        mv = vec * mask
        ex = jnp.exp(mv - jnp.max(mv)) * mask
        return ex / (jnp.sum(ex) + EPS)

    w_row = msm(a_row, mask_row)
    w_col = msm(a_col, mask_col)
    weff = (jnp.sum(jnp.where(sel_row, w_row[None, :], 0.0), axis=1)
            + jnp.sum(jnp.where(sel_col, w_col[None, :], 0.0), axis=1))
    ctx1 = jnp.dot((w_row + w_col)[None, :], phr_ref[...],
                   preferred_element_type=F32)
    ctx2 = jnp.dot(weff[None, :], upd_ref[...], preferred_element_type=F32)
    delta = (jnp.dot(ctx1, wpa_ref[...], preferred_element_type=F32)
             + jnp.dot(ctx2, wpb_ref[...], preferred_element_type=F32)
             + bphr_ref[...])
    row_is_t = jax.lax.broadcasted_iota(jnp.int32, (N, 1), 0) == t
    out_ref[...] = phr_ref[...] + jnp.where(row_is_t, delta, 0.0)


@jax.jit
def _run(phrase_feat, rel_feat, rel_conn_mat, target_id, W_rel, b_rel, W_sub,
         b_sub, W_obj, b_obj, W_phr, b_phr):
    sub = rel_conn_mat[0:1].astype(jnp.int32)
    obj = rel_conn_mat[1:2].astype(jnp.int32)
    t = jnp.asarray(target_id, jnp.int32).reshape(1, 1)
    wa, wb, wc = W_rel[:D], W_rel[D:2 * D], W_rel[2 * D:]
    wsa, wsb = W_sub[:D], W_sub[D:]
    woa, wob = W_obj[:D], W_obj[D:]
    wpa, wpb = W_phr[:D], W_phr[D:]
    brel = b_rel.reshape(1, D)
    bsub = b_sub.reshape(1, D)
    bobj = b_obj.reshape(1, D)
    bphr = b_phr.reshape(1, D)

    t_sub, t_obj = pl.pallas_call(
        _tables_body,
        out_shape=(jax.ShapeDtypeStruct((N, 2 * D), F32),
                   jax.ShapeDtypeStruct((N, 2 * D), F32)),
    )(phrase_feat, wa, wsa, wb, woa)

    nblk = E // EB
    upd, atte = pl.pallas_call(
        _edges_body,
        grid=(nblk,),
        in_specs=[
            pl.BlockSpec((1, EB), lambda i: (0, i)),
            pl.BlockSpec((1, EB), lambda i: (0, i)),
            pl.BlockSpec((EB, D), lambda i: (i, 0)),
            pl.BlockSpec((N, 2 * D), lambda i: (0, 0)),
            pl.BlockSpec((N, 2 * D), lambda i: (0, 0)),
            pl.BlockSpec((D, D), lambda i: (0, 0)),
            pl.BlockSpec((D, D), lambda i: (0, 0)),
            pl.BlockSpec((D, D), lambda i: (0, 0)),
            pl.BlockSpec((1, D), lambda i: (0, 0)),
            pl.BlockSpec((1, D), lambda i: (0, 0)),
            pl.BlockSpec((1, D), lambda i: (0, 0)),
        ],
        out_specs=(pl.BlockSpec((EB, D), lambda i: (i, 0)),
                   pl.BlockSpec((1, EB), lambda i: (0, i))),
        out_shape=(jax.ShapeDtypeStruct((E, D), F32),
                   jax.ShapeDtypeStruct((1, E), F32)),
    )(sub, obj, rel_feat, t_sub, t_obj, wc, wsb, wob, brel, bsub, bobj)

    out1 = pl.pallas_call(
        _context_body,
        out_shape=jax.ShapeDtypeStruct((N, D), F32),
    )(sub, obj, atte, upd, phrase_feat, wpa, wpb, bphr, t)
    return out1, upd


def kernel(phrase_feat, rel_feat, rel_conn_mat, target_id, W_rel, b_rel,
           W_sub, b_sub, W_obj, b_obj, W_phr, b_phr):
    return _run(phrase_feat, rel_feat, rel_conn_mat, target_id, W_rel, b_rel,
                W_sub, b_sub, W_obj, b_obj, W_phr, b_phr)


# speedup vs baseline: 35.9859x; 1.7208x over previous
import jax
import jax.numpy as jnp
from jax.experimental import pallas as pl
from jax.experimental.pallas import tpu as pltpu

N = 256
D = 256
E = 4096
EB = 1024
NBLK = E // EB
EPS = 1e-06
F32 = jnp.float32


def _fused_body(sub_ref, obj_ref, rel_ref, sub_all_ref, obj_all_ref, phr_ref,
                wrel_ref, wsub_ref, wobj_ref, wphr_ref, brel_ref, bsub_ref,
                bobj_ref, bphr_ref, t_ref, upd_ref, out_ref, tsub_s, tobj_s,
                atte_s):
    i = pl.program_id(0)

    @pl.when(i == 0)
    def _tables():
        phr = phr_ref[...]
        tsub_s[:, :D] = jnp.dot(phr, wrel_ref[:D], preferred_element_type=F32)
        tsub_s[:, D:] = jnp.dot(phr, wsub_ref[:D], preferred_element_type=F32)
        tobj_s[:, :D] = jnp.dot(phr, wrel_ref[D:2 * D],
                                preferred_element_type=F32)
        tobj_s[:, D:] = jnp.dot(phr, wobj_ref[:D], preferred_element_type=F32)

    sub = sub_ref[0, :]
    obj = obj_ref[0, :]
    iota_n = jax.lax.broadcasted_iota(jnp.int32, (EB, N), 1)
    oh_sub = (sub[:, None] == iota_n).astype(F32)
    oh_obj = (obj[:, None] == iota_n).astype(F32)
    gsub = jnp.dot(oh_sub, tsub_s[...], preferred_element_type=F32)
    gobj = jnp.dot(oh_obj, tobj_s[...], preferred_element_type=F32)
    upd = (gsub[:, :D] + gobj[:, :D] + brel_ref[...]
           + jnp.dot(rel_ref[...], wrel_ref[2 * D:],
                     preferred_element_type=F32))
    upd_ref[pl.ds(i * EB, EB), :] = upd
    ts = gsub[:, D:] + bsub_ref[...] + jnp.dot(upd, wsub_ref[D:],
                                               preferred_element_type=F32)
    to = gobj[:, D:] + bobj_ref[...] + jnp.dot(upd, wobj_ref[D:],
                                               preferred_element_type=F32)
    ones_col = jnp.ones((D, 1), dtype=F32)
    atte_s[pl.ds(i * EB, EB), :] = jnp.dot(ts * to, ones_col,
                                           preferred_element_type=F32) * (
                                               1.0 / (D ** 0.5))

    @pl.when(i == NBLK - 1)
    def _context():
        t = t_ref[0, 0]
        sub_all = sub_all_ref[0, :]
        obj_all = obj_all_ref[0, :]
        iota_e = jax.lax.broadcasted_iota(jnp.int32, (E, N), 0)
        iota_o = jax.lax.broadcasted_iota(jnp.int32, (E, N), 1)
        subc = sub_all[:, None]
        objc = obj_all[:, None]
        e_row = jnp.max(jnp.where((subc == t) & (objc == iota_o), iota_e, -1),
                        axis=0)
        e_col = jnp.max(jnp.where((objc == t) & (subc == iota_o), iota_e, -1),
                        axis=0)
        sel_row = (iota_e == e_row[None, :]).astype(F32)
        sel_col = (iota_e == e_col[None, :]).astype(F32)
        atte_col = atte_s[...]
        a_row = jax.lax.dot_general(atte_col, sel_row, (((0,), (0,)), ((), ())),
                                    preferred_element_type=F32)[0]
        a_col = jax.lax.dot_general(atte_col, sel_col, (((0,), (0,)), ((), ())),
                                    preferred_element_type=F32)[0]
        mask_row = (e_row >= 0).astype(F32)
        mask_col = (e_col >= 0).astype(F32)

        def msm(vec, mask):
            mv = vec * mask
            ex = jnp.exp(mv - jnp.max(mv)) * mask
            return ex / (jnp.sum(ex) + EPS)

        w_row = msm(a_row, mask_row)
        w_col = msm(a_col, mask_col)
        weff = (jnp.dot(sel_row, w_row[:, None], preferred_element_type=F32)
                + jnp.dot(sel_col, w_col[:, None], preferred_element_type=F32))
        ctx1 = jnp.dot((w_row + w_col)[None, :], phr_ref[...],
                       preferred_element_type=F32)
        ctx2 = jax.lax.dot_general(weff, upd_ref[...], (((0,), (0,)), ((), ())),
                                   preferred_element_type=F32)
        delta = (jnp.dot(ctx1, wphr_ref[:D], preferred_element_type=F32)
                 + jnp.dot(ctx2, wphr_ref[D:], preferred_element_type=F32)
                 + bphr_ref[...])
        row_is_t = jax.lax.broadcasted_iota(jnp.int32, (N, 1), 0) == t
        out_ref[...] = phr_ref[...] + jnp.where(row_is_t, delta, 0.0)


@jax.jit
def _run(phrase_feat, rel_feat, rel_conn_mat, target_id, W_rel, b_rel, W_sub,
         b_sub, W_obj, b_obj, W_phr, b_phr):
    sub = rel_conn_mat[0:1].astype(jnp.int32)
    obj = rel_conn_mat[1:2].astype(jnp.int32)
    t = jnp.asarray(target_id, jnp.int32).reshape(1, 1)
    brel = b_rel.reshape(1, D)
    bsub = b_sub.reshape(1, D)
    bobj = b_obj.reshape(1, D)
    bphr = b_phr.reshape(1, D)

    full = lambda shape: pl.BlockSpec(shape, lambda i: tuple(0 for _ in shape))
    upd, out1 = pl.pallas_call(
        _fused_body,
        grid=(NBLK,),
        in_specs=[
            pl.BlockSpec((1, EB), lambda i: (0, i)),
            pl.BlockSpec((1, EB), lambda i: (0, i)),
            pl.BlockSpec((EB, D), lambda i: (i, 0)),
            full((1, E)),
            full((1, E)),
            full((N, D)),
            full((3 * D, D)),
            full((2 * D, D)),
            full((2 * D, D)),
            full((2 * D, D)),
            full((1, D)),
            full((1, D)),
            full((1, D)),
            full((1, D)),
            full((1, 1)),
        ],
        out_specs=(full((E, D)), full((N, D))),
        out_shape=(jax.ShapeDtypeStruct((E, D), F32),
                   jax.ShapeDtypeStruct((N, D), F32)),
        scratch_shapes=[
            pltpu.VMEM((N, 2 * D), F32),
            pltpu.VMEM((N, 2 * D), F32),
            pltpu.VMEM((E, 1), F32),
        ],
    )(sub, obj, rel_feat, sub, obj, phrase_feat, W_rel, W_sub, W_obj, W_phr,
      brel, bsub, bobj, bphr, t)
    return out1, upd


def kernel(phrase_feat, rel_feat, rel_conn_mat, target_id, W_rel, b_rel,
           W_sub, b_sub, W_obj, b_obj, W_phr, b_phr):
    return _run(phrase_feat, rel_feat, rel_conn_mat, target_id, W_rel, b_rel,
                W_sub, b_sub, W_obj, b_obj, W_phr, b_phr)
